# baseline (speedup 1.0000x reference)
import dataclasses
import functools

import jax
import jax.numpy as jnp
from jax import lax
from jax.experimental import pallas as pl
from jax.experimental.pallas import tpu as pltpu
from jax.experimental.pallas import tpu_sc as plsc

B = 1024
D = 64
S = 8
L = 16
NC = 2
NS = 16
NW = NC * NS
BPW = B // NW
NP = BPW * S
CH = 32
NCH = NP // CH


def _sc_gather_pool(user_idx, item_idx, item_emb, user_emb, adj_item, adj_adam):
    mesh = plsc.VectorSubcoreMesh(core_axis_name="c", subcore_axis_name="s")
    out_type = (
        jax.ShapeDtypeStruct((B, D), jnp.float32),
        jax.ShapeDtypeStruct((B, D), jnp.float32),
        jax.ShapeDtypeStruct((B * S, D), jnp.float32),
        jax.ShapeDtypeStruct((B * S,), jnp.float32),
        jax.ShapeDtypeStruct((B * S, D), jnp.float32),
        jax.ShapeDtypeStruct((B * S * S,), jnp.float32),
    )
    scratch = [
        pltpu.VMEM((BPW,), jnp.int32),
        pltpu.VMEM((BPW,), jnp.int32),
        pltpu.VMEM((BPW, D), jnp.float32),
        pltpu.VMEM((BPW, D), jnp.float32),
        pltpu.VMEM((BPW, S), jnp.int32),
        pltpu.VMEM((BPW * S,), jnp.int32),
        pltpu.VMEM((BPW, S), jnp.float32),
        pltpu.VMEM((BPW * S,), jnp.float32),
        pltpu.VMEM((NP, S), jnp.int32),
        pltpu.VMEM((NP * S,), jnp.int32),
        pltpu.VMEM((NP, S), jnp.float32),
        pltpu.VMEM((NP * S,), jnp.float32),
        pltpu.VMEM((NP, D), jnp.float32),
        pltpu.VMEM((CH * S, D), jnp.float32),
        pltpu.VMEM((CH * S, D), jnp.float32),
        pltpu.VMEM((NP, D), jnp.float32),
        pltpu.SemaphoreType.DMA,
        pltpu.SemaphoreType.DMA,
        pltpu.SemaphoreType.DMA,
        pltpu.SemaphoreType.DMA,
        pltpu.SemaphoreType.DMA,
    ]

    cp = pltpu.CompilerParams(needs_layout_passes=False,
                              use_tc_tiling_on_sc=False)

    @functools.partial(pl.kernel, mesh=mesh, out_type=out_type,
                       scratch_types=scratch, compiler_params=cp)
    def k(uidx_h, iidx_h, item_h, user_h, adji_h, adja_h,
          u_o, iv0_o, iv1_o, ew1_o, acc2_o, ew2_o,
          uidx, iidx, urows, iv0rows, e1, e1f, a1, ew1f,
          e2, e2f, a2, ew2f, iv1rows, ivb0, ivb1, accb,
          sem_u, sem_i0, sem_iv1, sem_a, sem_b):
        wid = lax.axis_index("s") * NC + lax.axis_index("c")
        base = wid * BPW
        lane = lax.iota(jnp.int32, L)
        lrow = lane // S
        lcol = lane % S

        pltpu.sync_copy(uidx_h.at[pl.ds(base, BPW)], uidx)
        pltpu.sync_copy(iidx_h.at[pl.ds(base, BPW)], iidx)
        cp_u = pltpu.async_copy(user_h.at[uidx], urows, sem_u)
        cp_i0 = pltpu.async_copy(item_h.at[iidx], iv0rows, sem_i0)
        pltpu.sync_copy(adji_h.at[iidx], e1)
        pltpu.sync_copy(adja_h.at[iidx], a1)

        @pl.loop(0, (BPW * S) // L)
        def _(i):
            r = 2 * i + lrow
            e1f[pl.ds(i * L, L)] = plsc.load_gather(e1, [r, lcol])
            ew1f[pl.ds(i * L, L)] = jnp.exp(plsc.load_gather(a1, [r, lcol]))

        cp_iv1 = pltpu.async_copy(item_h.at[e1f], iv1rows, sem_iv1)
        pltpu.sync_copy(adji_h.at[e1f], e2)
        pltpu.sync_copy(adja_h.at[e1f], a2)

        @pl.loop(0, (NP * S) // L)
        def _(i):
            r = 2 * i + lrow
            e2f[pl.ds(i * L, L)] = plsc.load_gather(e2, [r, lcol])
            ew2f[pl.ds(i * L, L)] = jnp.exp(plsc.load_gather(a2, [r, lcol]))

        pltpu.sync_copy(ew1f, ew1_o.at[pl.ds(base * S, BPW * S)])
        pltpu.sync_copy(ew2f, ew2_o.at[pl.ds(base * S * S, NP * S)])
        cp_u.wait()
        pltpu.sync_copy(urows, u_o.at[pl.ds(base, BPW)])
        cp_i0.wait()
        pltpu.sync_copy(iv0rows, iv0_o.at[pl.ds(base, BPW)])
        cp_iv1.wait()
        pltpu.sync_copy(iv1rows, iv1_o.at[pl.ds(base * S, NP)])

        bufs = (ivb0, ivb1)
        sems = (sem_a, sem_b)
        cps = [None, None]
        cps[0] = pltpu.async_copy(
            item_h.at[e2f.at[pl.ds(0, CH * S)]], bufs[0], sems[0])
        for c in range(NCH):
            if c + 1 < NCH:
                cps[(c + 1) % 2] = pltpu.async_copy(
                    item_h.at[e2f.at[pl.ds((c + 1) * CH * S, CH * S)]],
                    bufs[(c + 1) % 2], sems[(c + 1) % 2])
            cps[c % 2].wait()
            buf = bufs[c % 2]

            @pl.loop(0, CH)
            def _(p, c=c, buf=buf):
                pair = c * CH + p
                wbase = pair * S
                ws = [
                    plsc.load_gather(
                        ew2f, [jnp.full((L,), wbase + s, jnp.int32)])
                    for s in range(S)
                ]
                for f in range(D // L):
                    acc = ws[0] * buf[p * S, pl.ds(f * L, L)]
                    for s in range(1, S):
                        acc = acc + ws[s] * buf[p * S + s, pl.ds(f * L, L)]
                    accb[pair, pl.ds(f * L, L)] = acc

        pltpu.sync_copy(accb, acc2_o.at[pl.ds(base * S, NP)])

    return k(user_idx, item_idx, item_emb, user_emb, adj_item, adj_adam)


def _tc_mlp(u, iv0, iv1, ew1, ew1f, acc2, ew2,
            pool_W, pool_b, fc1_W, fc1_b, fc2_W, fc2_b):
    BB = 256
    SD = jnp.float32

    def body(u_r, iv0_r, iv1_r, ew1_r, ew1f_r, acc2_r, ew2_r,
             pW_r, pb_r, f1W_r, f1b_r, f2W_r, f2b_r, o_r):
        dot = functools.partial(jnp.dot, preferred_element_type=SD,
                                precision=lax.Precision.HIGHEST)
        Wt = pW_r[:D, :]
        Wb = pW_r[D:, :]
        pb = pb_r[0, :]
        ew2 = ew2_r[...]
        den2 = jnp.sum(ew2, axis=1, keepdims=True)
        pooled2 = acc2_r[...] / (S * den2)
        iv1 = iv1_r[...]
        h1 = jnp.maximum(dot(iv1, Wt) + dot(pooled2, Wb) + pb, 0.0)
        den1 = jnp.sum(ew1_r[...], axis=1, keepdims=True)
        w1f = ew1f_r[...]
        pooled1 = jnp.sum((w1f * iv1).reshape(BB, S, D), axis=1) / (S * den1)
        h0 = jnp.maximum(dot(iv0_r[...], Wt) + dot(pooled1, Wb) + pb, 0.0)
        hp = jnp.sum((w1f * h1).reshape(BB, S, D), axis=1) / (S * den1)
        fin = jnp.maximum(dot(h0, Wt) + dot(hp, Wb) + pb, 0.0)
        z = jnp.maximum(dot(u_r[...], f1W_r[:D, :]) + dot(fin, f1W_r[D:, :])
                        + f1b_r[0, :], 0.0)
        o_r[...] = dot(z, f2W_r[...]) + f2b_r[0, :]

    return pl.pallas_call(
        body,
        grid=(B // BB,),
        in_specs=[
            pl.BlockSpec((BB, D), lambda i: (i, 0)),
            pl.BlockSpec((BB, D), lambda i: (i, 0)),
            pl.BlockSpec((BB * S, D), lambda i: (i, 0)),
            pl.BlockSpec((BB, S), lambda i: (i, 0)),
            pl.BlockSpec((BB * S, 1), lambda i: (i, 0)),
            pl.BlockSpec((BB * S, D), lambda i: (i, 0)),
            pl.BlockSpec((BB * S, S), lambda i: (i, 0)),
            pl.BlockSpec((2 * D, D), lambda i: (0, 0)),
            pl.BlockSpec((1, D), lambda i: (0, 0)),
            pl.BlockSpec((2 * D, 32), lambda i: (0, 0)),
            pl.BlockSpec((1, 32), lambda i: (0, 0)),
            pl.BlockSpec((32, 2), lambda i: (0, 0)),
            pl.BlockSpec((1, 2), lambda i: (0, 0)),
        ],
        out_specs=pl.BlockSpec((BB, 2), lambda i: (i, 0)),
        out_shape=jax.ShapeDtypeStruct((B, 2), jnp.float32),
    )(u, iv0, iv1, ew1, ew1f, acc2, ew2, pool_W, pool_b.reshape(1, D),
      fc1_W, fc1_b.reshape(1, 32), fc2_W, fc2_b.reshape(1, 2))


def kernel(inputs, item_emb, user_emb, adj_item, adj_adam,
           pool_W, pool_b, fc1_W, fc1_b, fc2_W, fc2_b):
    user_idx = inputs[:, 0].astype(jnp.int32)
    item_idx = inputs[:, 1].astype(jnp.int32)
    u, iv0, iv1, ew1f, acc2, ew2f = _sc_gather_pool(
        user_idx, item_idx, item_emb, user_emb,
        adj_item.astype(jnp.int32), adj_adam)
    ew1 = ew1f.reshape(B, S)
    ew2 = ew2f.reshape(B * S, S)
    return _tc_mlp(u, iv0, iv1, ew1, ew1f.reshape(B * S, 1), acc2, ew2,
                   pool_W, pool_b, fc1_W, fc1_b, fc2_W, fc2_b)
